# speedup vs baseline: 1.0961x; 1.0961x over previous
import functools

import jax
import jax.numpy as jnp
from jax import lax
from jax.experimental import pallas as pl
from jax.experimental.pallas import tpu as pltpu
from jax.experimental.pallas import tpu_sc as plsc

NUM_CORES = 2
NUM_SUBCORES = 16
NUM_WORKERS = NUM_CORES * NUM_SUBCORES
LANES = 16
CHUNK = 32


def _emb_body(B, T, D, ids_hbm, tok_hbm, pos_hbm, out_hbm, idx_v, tok_v, pos_v, sem):
    t_per_w = T // NUM_WORKERS
    wid = lax.axis_index("s") * NUM_CORES + lax.axis_index("c")
    t0 = wid * t_per_w
    for h in range(t_per_w // CHUNK):
        pos_base = t0 + h * CHUNK
        pltpu.sync_copy(pos_hbm.at[pl.ds(pos_base, CHUNK)], pos_v)
        for b in range(B):
            flat_base = b * T + pos_base
            pltpu.sync_copy(ids_hbm.at[pl.ds(flat_base, CHUNK)], idx_v)
            pltpu.async_copy(tok_hbm.at[idx_v], tok_v, sem).wait()

            @pl.loop(0, CHUNK)
            def _row(r):
                @pl.loop(0, D, step=LANES, unroll=4)
                def _col(c):
                    tok_v[r, pl.ds(c, LANES)] = (
                        tok_v[r, pl.ds(c, LANES)] + pos_v[r, pl.ds(c, LANES)]
                    )

            pltpu.sync_copy(tok_v, out_hbm.at[pl.ds(flat_base, CHUNK)])


def kernel(input_ids, tok_table, pos_table):
    B, T = input_ids.shape
    D = tok_table.shape[1]
    ids_flat = input_ids.reshape(-1).astype(jnp.int32)
    mesh = plsc.VectorSubcoreMesh(core_axis_name="c", subcore_axis_name="s")
    k = pl.kernel(
        functools.partial(_emb_body, B, T, D),
        out_type=jax.ShapeDtypeStruct((B * T, D), jnp.float32),
        mesh=mesh,
        scratch_types=[
            pltpu.VMEM((CHUNK,), jnp.int32),
            pltpu.VMEM((CHUNK, D), jnp.float32),
            pltpu.VMEM((CHUNK, D), jnp.float32),
            pltpu.SemaphoreType.DMA,
        ],
    )
    out = k(ids_flat, tok_table, pos_table)
    return out.reshape(B, T, D)


# speedup vs baseline: 1.5364x; 1.4017x over previous
import functools

import jax
import jax.numpy as jnp
from jax import lax
from jax.experimental import pallas as pl
from jax.experimental.pallas import tpu as pltpu
from jax.experimental.pallas import tpu_sc as plsc

NUM_CORES = 2
NUM_SUBCORES = 16
NUM_WORKERS = NUM_CORES * NUM_SUBCORES
LANES = 16
CHUNK = 16
TBUF = 3
PBUF = 2


def _emb_body(B, T, D, ids_hbm, tok_hbm, pos_hbm, out_hbm, *rest):
    idx_v = rest[0]
    tok_bufs = rest[1:1 + TBUF]
    pos_bufs = rest[1 + TBUF:1 + TBUF + PBUF]
    gsems = rest[1 + TBUF + PBUF:1 + 2 * TBUF + PBUF]
    osems = rest[1 + 2 * TBUF + PBUF:1 + 3 * TBUF + PBUF]
    psems = rest[1 + 3 * TBUF + PBUF:1 + 3 * TBUF + 2 * PBUF]

    t_per_w = T // NUM_WORKERS
    nh = t_per_w // CHUNK
    nch = nh * B
    wid = lax.axis_index("s") * NUM_CORES + lax.axis_index("c")
    t0 = wid * t_per_w

    for b in range(B):
        pltpu.sync_copy(ids_hbm.at[pl.ds(b * T + t0, t_per_w)], idx_v.at[b])

    pos_d = [None] * nh
    gat = [None] * nch
    out = [None] * nch
    pos_d[0] = pltpu.async_copy(
        pos_hbm.at[pl.ds(t0, CHUNK)], pos_bufs[0], psems[0]
    )
    for t in range(nch + 1):
        g = t
        if g < nch:
            if g >= TBUF:
                out[g - TBUF].wait()
            h, b = divmod(g, B)
            gat[g] = pltpu.async_copy(
                tok_hbm.at[idx_v.at[b, pl.ds(h * CHUNK, CHUNK)]],
                tok_bufs[g % TBUF],
                gsems[g % TBUF],
            )
        g = t - 1
        if 0 <= g < nch:
            h, b = divmod(g, B)
            if b == 0:
                pos_d[h].wait()
                if h + 1 < nh:
                    pos_d[h + 1] = pltpu.async_copy(
                        pos_hbm.at[pl.ds(t0 + (h + 1) * CHUNK, CHUNK)],
                        pos_bufs[(h + 1) % PBUF],
                        psems[(h + 1) % PBUF],
                    )
            gat[g].wait()
            tok_v = tok_bufs[g % TBUF]
            pos_v = pos_bufs[h % PBUF]

            @pl.loop(0, CHUNK)
            def _row(r):
                @pl.loop(0, D, step=LANES, unroll=4)
                def _col(c):
                    tok_v[r, pl.ds(c, LANES)] = (
                        tok_v[r, pl.ds(c, LANES)] + pos_v[r, pl.ds(c, LANES)]
                    )

            out[g] = pltpu.async_copy(
                tok_v,
                out_hbm.at[pl.ds(b * T + t0 + h * CHUNK, CHUNK)],
                osems[g % TBUF],
            )
    for g in range(max(0, nch - TBUF), nch):
        out[g].wait()


def kernel(input_ids, tok_table, pos_table):
    B, T = input_ids.shape
    D = tok_table.shape[1]
    ids = input_ids.reshape(-1).astype(jnp.int32)
    t_per_w = T // NUM_WORKERS
    mesh = plsc.VectorSubcoreMesh(core_axis_name="c", subcore_axis_name="s")
    scratch = [pltpu.VMEM((B, t_per_w), jnp.int32)]
    scratch += [pltpu.VMEM((CHUNK, D), jnp.float32) for _ in range(TBUF)]
    scratch += [pltpu.VMEM((CHUNK, D), jnp.float32) for _ in range(PBUF)]
    scratch += [pltpu.SemaphoreType.DMA for _ in range(2 * TBUF + 2 * PBUF)]
    k = pl.kernel(
        functools.partial(_emb_body, B, T, D),
        out_type=jax.ShapeDtypeStruct((B * T, D), jnp.float32),
        mesh=mesh,
        scratch_types=scratch,
    )
    out = k(ids, tok_table, pos_table)
    return out.reshape(B, T, D)


# speedup vs baseline: 1.5421x; 1.0037x over previous
import functools

import jax
import jax.numpy as jnp
from jax import lax
from jax.experimental import pallas as pl
from jax.experimental.pallas import tpu as pltpu
from jax.experimental.pallas import tpu_sc as plsc

NUM_CORES = 2
NUM_SUBCORES = 16
NUM_WORKERS = NUM_CORES * NUM_SUBCORES
LANES = 16
CHUNK = 16
TBUF = 3
PBUF = 2


def _emb_body(B, T, D, ids_hbm, tok_hbm, pos_hbm, out_hbm, *rest):
    idx_v = rest[0]
    tok_bufs = rest[1:1 + TBUF]
    pos_bufs = rest[1 + TBUF:1 + TBUF + PBUF]
    gsems = rest[1 + TBUF + PBUF:1 + 2 * TBUF + PBUF]
    osems = rest[1 + 2 * TBUF + PBUF:1 + 3 * TBUF + PBUF]
    psems = rest[1 + 3 * TBUF + PBUF:1 + 3 * TBUF + 2 * PBUF]

    t_per_w = T // NUM_WORKERS
    nh = t_per_w // CHUNK
    nch = nh * B
    wid = lax.axis_index("s") * NUM_CORES + lax.axis_index("c")
    t0 = wid * t_per_w

    for b in range(B):
        pltpu.sync_copy(ids_hbm.at[pl.ds(b * T + t0, t_per_w)], idx_v.at[b])

    pos_d = [None] * nh
    gat = [None] * nch
    out = [None] * nch
    pos_d[0] = pltpu.async_copy(
        pos_hbm.at[pl.ds(t0, CHUNK)], pos_bufs[0], psems[0]
    )
    for t in range(nch + 1):
        g = t
        if g < nch:
            if g >= TBUF:
                out[g - TBUF].wait()
            h, b = divmod(g, B)
            gat[g] = pltpu.async_copy(
                tok_hbm.at[idx_v.at[b, pl.ds(h * CHUNK, CHUNK)]],
                tok_bufs[g % TBUF],
                gsems[g % TBUF],
            )
        g = t - 1
        if 0 <= g < nch:
            h, b = divmod(g, B)
            if b == 0:
                pos_d[h].wait()
                if h + 1 < nh:
                    pos_d[h + 1] = pltpu.async_copy(
                        pos_hbm.at[pl.ds(t0 + (h + 1) * CHUNK, CHUNK)],
                        pos_bufs[(h + 1) % PBUF],
                        psems[(h + 1) % PBUF],
                    )
            gat[g].wait()
            tok_v = tok_bufs[g % TBUF]
            pos_v = pos_bufs[h % PBUF]

            @pl.loop(0, CHUNK)
            def _row(r):
                @plsc.parallel_loop(0, D, step=LANES, unroll=8)
                def _col(c):
                    tok_v[r, pl.ds(c, LANES)] = (
                        tok_v[r, pl.ds(c, LANES)] + pos_v[r, pl.ds(c, LANES)]
                    )

            out[g] = pltpu.async_copy(
                tok_v,
                out_hbm.at[pl.ds(b * T + t0 + h * CHUNK, CHUNK)],
                osems[g % TBUF],
            )
    for g in range(max(0, nch - TBUF), nch):
        out[g].wait()


def kernel(input_ids, tok_table, pos_table):
    B, T = input_ids.shape
    D = tok_table.shape[1]
    ids = input_ids.reshape(-1).astype(jnp.int32)
    t_per_w = T // NUM_WORKERS
    mesh = plsc.VectorSubcoreMesh(core_axis_name="c", subcore_axis_name="s")
    scratch = [pltpu.VMEM((B, t_per_w), jnp.int32)]
    scratch += [pltpu.VMEM((CHUNK, D), jnp.float32) for _ in range(TBUF)]
    scratch += [pltpu.VMEM((CHUNK, D), jnp.float32) for _ in range(PBUF)]
    scratch += [pltpu.SemaphoreType.DMA for _ in range(2 * TBUF + 2 * PBUF)]
    k = pl.kernel(
        functools.partial(_emb_body, B, T, D),
        out_type=jax.ShapeDtypeStruct((B * T, D), jnp.float32),
        mesh=mesh,
        scratch_types=scratch,
    )
    out = k(ids, tok_table, pos_table)
    return out.reshape(B, T, D)
